# baseline (speedup 1.0000x reference)
import functools

import jax
import jax.numpy as jnp
from jax import lax
from jax.experimental import pallas as pl
from jax.experimental.pallas import tpu as pltpu
from jax.experimental.pallas import tpu_sc as plsc

_N = 10000
_NP = 10240
_E = 320000
_D = 128
_NC = 2
_NS = 16
_CH = 80
_NB = 3


def _make_sc_agg():
    mesh = plsc.VectorSubcoreMesh(core_axis_name="c", subcore_axis_name="s")
    n_workers = _NC * _NS
    epw = _E // n_workers
    n_chunks = epw // _CH
    rows_per_tile = _NP // _NS

    rpt0 = 624
    rpt1 = _N - (_NS - 1) * rpt0

    @functools.partial(
        pl.kernel,
        mesh=mesh,
        out_type=jax.ShapeDtypeStruct((_NC, _N, _D), jnp.float32),
        scratch_types=[
            pltpu.VMEM((epw,), jnp.int32),
            pltpu.VMEM((_CH,), jnp.int32),
            pltpu.VMEM((_CH,), jnp.int32),
            pltpu.VMEM((_CH,), jnp.int32),
            pltpu.VMEM((_NB, _CH, _D), jnp.float32),
            pltpu.VMEM_SHARED((_N, _D), jnp.float32),
            pltpu.SemaphoreType.DMA,
            pltpu.SemaphoreType.DMA,
        ] + [pltpu.SemaphoreType.DMA] * (3 * _NB),
    )
    def sc_agg(x_hbm, src_hbm, dst_hbm, zeros_hbm, out_hbm,
               idx_v, dstc0, dstc1, dstc2, rows, agg_sh, zsem, isem, *sems):
        dstcs = [dstc0, dstc1, dstc2]
        gsems = list(sems[:_NB])
        dsems = list(sems[_NB:2 * _NB])
        ssems = list(sems[2 * _NB:])
        cid = lax.axis_index("c")
        sid = lax.axis_index("s")
        wid = sid * _NC + cid
        last = sid == _NS - 1
        r0 = sid * rpt0

        def rng(ref):
            return (ref.at[pl.ds(r0, rpt0)],
                    ref.at[pl.ds((_NS - 1) * rpt0, rpt1)])

        zsrc0, zsrc1 = rng(zeros_hbm)
        zdst0, zdst1 = rng(agg_sh)

        @pl.when(~last)
        def _():
            pltpu.async_copy(zsrc0, zdst0, zsem)

        @pl.when(last)
        def _():
            pltpu.async_copy(zsrc1, zdst1, zsem)

        base = wid * epw
        pltpu.async_copy(src_hbm.at[pl.ds(base, epw)], idx_v, isem)

        @pl.when(~last)
        def _():
            pltpu.make_async_copy(zsrc0, zdst0, zsem).wait()

        @pl.when(last)
        def _():
            pltpu.make_async_copy(zsrc1, zdst1, zsem).wait()

        pltpu.make_async_copy(src_hbm.at[pl.ds(base, epw)], idx_v, isem).wait()
        plsc.subcore_barrier()

        def sl(c):
            return pl.ds(pl.multiple_of(c * _CH, _CH), _CH)

        def issue(c, k):
            pltpu.async_copy(
                dst_hbm.at[pl.ds(pl.multiple_of(base + c * _CH, _CH), _CH)],
                dstcs[k], dsems[k])
            pltpu.async_copy(x_hbm.at[idx_v.at[sl(c)]], rows.at[k], gsems[k])

        def wait_scatter(k):
            pltpu.make_async_copy(rows.at[k], agg_sh.at[dstcs[k]],
                                  ssems[k]).wait()

        def step(c, k, guard_lo, issue_next):
            if guard_lo:
                @pl.when(c >= 2)
                def _():
                    wait_scatter((k + 1) % _NB)
            else:
                wait_scatter((k + 1) % _NB)
            if issue_next:
                issue(c + 1, (k + 1) % _NB)
            pltpu.make_async_copy(x_hbm.at[idx_v.at[sl(0)]], rows.at[k],
                                  gsems[k]).wait()
            pltpu.make_async_copy(dst_hbm.at[pl.ds(0, _CH)], dstcs[k],
                                  dsems[k]).wait()
            pltpu.async_copy(rows.at[k], agg_sh.at[dstcs[k]], ssems[k],
                             add=True)

        issue(0, 0)

        def body(i, carry):
            c0 = i * _NB
            for k in range(_NB):
                step(c0 + k, k, True, True)
            return carry

        lax.fori_loop(0, n_chunks // _NB, body, 0)
        ntail = n_chunks - _NB * (n_chunks // _NB)
        for t in range(ntail):
            step(n_chunks - ntail + t, t, False, t < ntail - 1)
        wait_scatter((n_chunks - 2) % _NB)
        wait_scatter((n_chunks - 1) % _NB)
        plsc.subcore_barrier()

        @pl.when(~last)
        def _():
            pltpu.sync_copy(agg_sh.at[pl.ds(r0, rpt0)],
                            out_hbm.at[cid, pl.ds(r0, rpt0)])

        @pl.when(last)
        def _():
            pltpu.sync_copy(agg_sh.at[pl.ds((_NS - 1) * rpt0, rpt1)],
                            out_hbm.at[cid, pl.ds((_NS - 1) * rpt0, rpt1)])

    return sc_agg


_sc_agg = _make_sc_agg()

_BLK = 1000


def _mlp_body(x_ref, a_ref, w1_ref, b1_ref, w2_ref, b2_ref, o_ref):
    h = x_ref[...] + a_ref[0] + a_ref[1]
    h = jnp.dot(h, w1_ref[...], preferred_element_type=jnp.float32) + b1_ref[...]
    h = jnp.maximum(h, 0.01 * h)
    h = jnp.dot(h, w2_ref[...], preferred_element_type=jnp.float32) + b2_ref[...]
    o_ref[...] = jnp.maximum(h, 0.01 * h)


def _tc_mlp(x, agg2, W1, b1, W2, b2):
    return pl.pallas_call(
        _mlp_body,
        grid=(_N // _BLK,),
        in_specs=[
            pl.BlockSpec((_BLK, _D), lambda i: (i, 0)),
            pl.BlockSpec((_NC, _BLK, _D), lambda i: (0, i, 0)),
            pl.BlockSpec((_D, _D), lambda i: (0, 0)),
            pl.BlockSpec((1, _D), lambda i: (0, 0)),
            pl.BlockSpec((_D, _D), lambda i: (0, 0)),
            pl.BlockSpec((1, _D), lambda i: (0, 0)),
        ],
        out_specs=pl.BlockSpec((_BLK, _D), lambda i: (i, 0)),
        out_shape=jax.ShapeDtypeStruct((_N, _D), jnp.float32),
    )(x, agg2, W1, b1.reshape(1, _D), W2, b2.reshape(1, _D))


def kernel(x, edge_index, W1, b1, W2, b2):
    src = edge_index[0]
    dst = edge_index[1]
    zeros = jnp.zeros((_N, _D), jnp.float32)
    agg2 = _sc_agg(x, src, dst, zeros)
    return _tc_mlp(x, agg2, W1, b1, W2, b2)
